# baseline (speedup 1.0000x reference)
import numpy as np
import jax
import jax.numpy as jnp
from jax.experimental import pallas as pl
from jax.experimental.pallas import tpu as pltpu

_B, _L, _T, _OBS, _ACT = 8192, 5, 16, 64, 16
_BLK = 256


def _ahat_np():
    edges = [[i, i + 1] for i in range(_T - 1)] + [[1, 6], [2, 5]]
    a = np.eye(_T, dtype=np.float64)
    for s, d in edges:
        a[s, d] = 1.0
        a[d, s] = 1.0
    deg = a.sum(axis=1)
    dinv = 1.0 / np.sqrt(deg)
    return (dinv[:, None] * a * dinv[None, :]).astype(np.float32)


_AHAT = _ahat_np()
_NBRS = [[(j, float(_AHAT[i, j])) for j in range(_T) if _AHAT[i, j] != 0.0]
         for i in range(_T)]


def _agg(x, w):
    cols = []
    for i in range(_T):
        a = None
        for j, c in _NBRS[i]:
            v = x[:, j * w:(j + 1) * w] * c
            a = v if a is None else a + v
        cols.append(a)
    return jnp.concatenate(cols, axis=1)


def _body(d_ref, wp_ref, bp_ref, out_ref, dscr):
    bf16 = jnp.bfloat16

    def mm(w, x):
        return jax.lax.dot_general(
            w, x, (((1,), (0,)), ((), ())), preferred_element_type=jnp.float32)

    w1 = wp_ref[0:64, :]
    w2 = wp_ref[64:128, :]
    wg1 = wp_ref[128:256, 0:64]
    wg2 = wp_ref[256:384, 0:128]
    wl = wp_ref[384:400, 0:128]
    b1 = bp_ref[0:192]
    b2 = bp_ref[192:256]
    bg1 = bp_ref[256:384]
    bg2 = bp_ref[384:512]
    bl = bp_ref[512:528]

    for l in range(_L):
        for t in range(_T):
            dscr[64 * l:64 * (l + 1), t * _BLK:(t + 1) * _BLK] = (
                d_ref[l, t].astype(bf16))

    h = jnp.concatenate(
        [mm(w1, dscr[64 * p:64 * p + 192, :]) for p in range(3)], axis=0)
    h = jax.nn.relu(h + b1).astype(bf16)
    z = jax.nn.relu(mm(w2, h) + b2).astype(bf16)
    g1 = jax.nn.relu(mm(wg1, _agg(z, _BLK)) + bg1).astype(bf16)
    g2 = jax.nn.relu(mm(wg2, _agg(g1, _BLK)) + bg2).astype(bf16)
    y = jnp.tanh(mm(wl, g2) + bl)
    pooled = None
    for t in range(_T):
        s = y[:, t * _BLK:(t + 1) * _BLK]
        pooled = s if pooled is None else pooled + s
    out_ref[...] = pooled * (1.0 / _T)


def kernel(data, W1, b1, W2, b2, Wg1, bg1, Wg2, bg2, Wl, bl):
    f32 = jnp.float32
    dt = jnp.transpose(data, (1, 2, 3, 0))

    w1cat = jnp.transpose(W1, (0, 2, 1)).reshape(64, 192)
    w2cat = jnp.transpose(W2, (0, 2, 1)).reshape(64, 192)
    padc = lambda a: jnp.pad(a, ((0, 0), (0, 192 - a.shape[1])))
    wpack = jnp.concatenate([
        w1cat, w2cat, padc(jnp.transpose(Wg1)), padc(jnp.transpose(Wg2)),
        padc(jnp.transpose(Wl))], axis=0).astype(jnp.bfloat16)
    bpack = jnp.concatenate([b1, b1, b1, b2, bg1, bg2, bl]).reshape(528, 1)

    full = lambda *shape: pl.BlockSpec(shape, lambda i: (0,) * len(shape))
    grid = (_B // _BLK,)
    out = pl.pallas_call(
        _body,
        grid=grid,
        in_specs=[
            pl.BlockSpec((_L, _T, _OBS, _BLK), lambda i: (0, 0, 0, i)),
            full(400, 192), full(528, 1),
        ],
        out_specs=pl.BlockSpec((_ACT, _BLK), lambda i: (0, i)),
        out_shape=jax.ShapeDtypeStruct((_ACT, _B), f32),
        scratch_shapes=[pltpu.VMEM((_L * _OBS, _T * _BLK), jnp.bfloat16)],
        compiler_params=pltpu.CompilerParams(
            dimension_semantics=("parallel",)),
    )(dt, wpack, bpack)
    return jnp.transpose(out)


# speedup vs baseline: 1.0753x; 1.0753x over previous
import numpy as np
import jax
import jax.numpy as jnp
from jax.experimental import pallas as pl
from jax.experimental.pallas import tpu as pltpu

_B, _L, _T, _OBS, _ACT = 8192, 5, 16, 64, 16
_BLK = 512


def _ahat_np():
    edges = [[i, i + 1] for i in range(_T - 1)] + [[1, 6], [2, 5]]
    a = np.eye(_T, dtype=np.float64)
    for s, d in edges:
        a[s, d] = 1.0
        a[d, s] = 1.0
    deg = a.sum(axis=1)
    dinv = 1.0 / np.sqrt(deg)
    return (dinv[:, None] * a * dinv[None, :]).astype(np.float32)


_AHAT = _ahat_np()
_NBRS = [[(j, float(_AHAT[i, j])) for j in range(_T) if _AHAT[i, j] != 0.0]
         for i in range(_T)]


def _agg(x, w):
    cols = []
    for i in range(_T):
        a = None
        for j, c in _NBRS[i]:
            v = x[:, j * w:(j + 1) * w] * c
            a = v if a is None else a + v
        cols.append(a)
    return jnp.concatenate(cols, axis=1)


def _body(d_ref, wp_ref, bp_ref, out_ref, dscr):
    bf16 = jnp.bfloat16

    def mm(w, x):
        return jax.lax.dot_general(
            w, x, (((1,), (0,)), ((), ())), preferred_element_type=jnp.float32)

    w1 = wp_ref[0:64, :]
    w2 = wp_ref[64:128, :]
    wg1 = wp_ref[128:256, 0:64]
    wg2 = wp_ref[256:384, 0:128]
    wl = wp_ref[384:400, 0:128]
    b1 = bp_ref[0:192]
    b2 = bp_ref[192:256]
    bg1 = bp_ref[256:384]
    bg2 = bp_ref[384:512]
    bl = bp_ref[512:528]

    for l in range(_L):
        for t in range(_T):
            dscr[64 * l:64 * (l + 1), t * _BLK:(t + 1) * _BLK] = (
                d_ref[l, t].astype(bf16))

    h = jnp.concatenate(
        [mm(w1, dscr[64 * p:64 * p + 192, :]) for p in range(3)], axis=0)
    h = jax.nn.relu(h + b1).astype(bf16)
    z = jax.nn.relu(mm(w2, h) + b2).astype(bf16)
    g1 = jax.nn.relu(mm(wg1, _agg(z, _BLK)) + bg1).astype(bf16)
    g2 = jax.nn.relu(mm(wg2, _agg(g1, _BLK)) + bg2).astype(bf16)
    y = jnp.tanh(mm(wl, g2) + bl)
    pooled = None
    for t in range(_T):
        s = y[:, t * _BLK:(t + 1) * _BLK]
        pooled = s if pooled is None else pooled + s
    out_ref[...] = pooled * (1.0 / _T)


def kernel(data, W1, b1, W2, b2, Wg1, bg1, Wg2, bg2, Wl, bl):
    f32 = jnp.float32
    dt = jnp.transpose(data, (1, 2, 3, 0))

    w1cat = jnp.transpose(W1, (0, 2, 1)).reshape(64, 192)
    w2cat = jnp.transpose(W2, (0, 2, 1)).reshape(64, 192)
    padc = lambda a: jnp.pad(a, ((0, 0), (0, 192 - a.shape[1])))
    wpack = jnp.concatenate([
        w1cat, w2cat, padc(jnp.transpose(Wg1)), padc(jnp.transpose(Wg2)),
        padc(jnp.transpose(Wl))], axis=0).astype(jnp.bfloat16)
    bpack = jnp.concatenate([b1, b1, b1, b2, bg1, bg2, bl]).reshape(528, 1)

    full = lambda *shape: pl.BlockSpec(shape, lambda i: (0,) * len(shape))
    grid = (_B // _BLK,)
    out = pl.pallas_call(
        _body,
        grid=grid,
        in_specs=[
            pl.BlockSpec((_L, _T, _OBS, _BLK), lambda i: (0, 0, 0, i)),
            full(400, 192), full(528, 1),
        ],
        out_specs=pl.BlockSpec((_ACT, _BLK), lambda i: (0, i)),
        out_shape=jax.ShapeDtypeStruct((_ACT, _B), f32),
        scratch_shapes=[pltpu.VMEM((_L * _OBS, _T * _BLK), jnp.bfloat16)],
        compiler_params=pltpu.CompilerParams(
            dimension_semantics=("parallel",)),
    )(dt, wpack, bpack)
    return jnp.transpose(out)


# speedup vs baseline: 1.0892x; 1.0130x over previous
import numpy as np
import jax
import jax.numpy as jnp
from jax.experimental import pallas as pl
from jax.experimental.pallas import tpu as pltpu

_B, _L, _T, _OBS, _ACT = 8192, 5, 16, 64, 16
_BLK = 512


def _ahat_np():
    edges = [[i, i + 1] for i in range(_T - 1)] + [[1, 6], [2, 5]]
    a = np.eye(_T, dtype=np.float64)
    for s, d in edges:
        a[s, d] = 1.0
        a[d, s] = 1.0
    deg = a.sum(axis=1)
    dinv = 1.0 / np.sqrt(deg)
    return (dinv[:, None] * a * dinv[None, :]).astype(np.float32)


_AHAT = _ahat_np()
_NBRS = [[(j, float(_AHAT[i, j])) for j in range(_T) if _AHAT[i, j] != 0.0]
         for i in range(_T)]


def _agg(x, w):
    cols = []
    for i in range(_T):
        a = None
        for j, c in _NBRS[i]:
            v = x[:, j * w:(j + 1) * w] * c
            a = v if a is None else a + v
        cols.append(a)
    return jnp.concatenate(cols, axis=1)


def _body(d_ref, wp_ref, bp_ref, wg1_ref, wg2_ref, wl_ref, out_ref, dscr):
    bf16 = jnp.bfloat16

    def mm(w, x):
        return jax.lax.dot_general(
            w, x, (((1,), (0,)), ((), ())), preferred_element_type=jnp.float32)

    def mmT(w, x):
        return jax.lax.dot_general(
            w, x, (((0,), (0,)), ((), ())), preferred_element_type=jnp.float32)

    w1 = wp_ref[0:64, :]
    w2 = wp_ref[64:128, :]
    wg1 = wg1_ref[...].astype(bf16)
    wg2 = wg2_ref[...].astype(bf16)
    wl = wl_ref[...].astype(bf16)
    b1 = bp_ref[0:192]
    b2 = bp_ref[192:256]
    bg1 = bp_ref[256:384]
    bg2 = bp_ref[384:512]
    bl = bp_ref[512:528]

    for l in range(_L):
        for t in range(_T):
            dscr[64 * l:64 * (l + 1), t * _BLK:(t + 1) * _BLK] = (
                d_ref[l, t].astype(bf16))

    h = jnp.concatenate(
        [mm(w1, dscr[64 * p:64 * p + 192, :]) for p in range(3)], axis=0)
    h = jax.nn.relu(h + b1).astype(bf16)
    z = jax.nn.relu(mm(w2, h) + b2).astype(bf16)
    g1 = jax.nn.relu(mmT(wg1, _agg(z, _BLK)) + bg1).astype(bf16)
    g2 = jax.nn.relu(mmT(wg2, _agg(g1, _BLK)) + bg2).astype(bf16)
    y = jnp.tanh(mmT(wl, g2) + bl)
    pooled = None
    for t in range(_T):
        s = y[:, t * _BLK:(t + 1) * _BLK]
        pooled = s if pooled is None else pooled + s
    out_ref[...] = pooled * (1.0 / _T)


def kernel(data, W1, b1, W2, b2, Wg1, bg1, Wg2, bg2, Wl, bl):
    f32 = jnp.float32
    dt = jnp.transpose(data, (1, 2, 3, 0))

    w1cat = jnp.transpose(W1, (0, 2, 1)).reshape(64, 192)
    w2cat = jnp.transpose(W2, (0, 2, 1)).reshape(64, 192)
    wpack = jnp.concatenate([w1cat, w2cat],
                            axis=0).astype(jnp.bfloat16)
    bpack = jnp.concatenate([b1, b1, b1, b2, bg1, bg2, bl]).reshape(528, 1)

    full = lambda *shape: pl.BlockSpec(shape, lambda i: (0,) * len(shape))
    grid = (_B // _BLK,)
    out = pl.pallas_call(
        _body,
        grid=grid,
        in_specs=[
            pl.BlockSpec((_L, _T, _OBS, _BLK), lambda i: (0, 0, 0, i)),
            full(128, 192), full(528, 1),
            full(64, 128), full(128, 128), full(128, _ACT),
        ],
        out_specs=pl.BlockSpec((_ACT, _BLK), lambda i: (0, i)),
        out_shape=jax.ShapeDtypeStruct((_ACT, _B), f32),
        scratch_shapes=[pltpu.VMEM((_L * _OBS, _T * _BLK), jnp.bfloat16)],
        compiler_params=pltpu.CompilerParams(
            dimension_semantics=("parallel",)),
    )(dt, wpack, bpack, Wg1, Wg2, Wl)
    return jnp.transpose(out)
